# baseline (speedup 1.0000x reference)
import functools

import jax
import jax.numpy as jnp
from jax import lax
from jax.experimental import pallas as pl
from jax.experimental.pallas import tpu as pltpu
from jax.experimental.pallas import tpu_sc as plsc

NUM_STATES = 128
NUM_OBS = 100000
BATCH = 16384

COL_BLK = 512
NBLK = (NUM_OBS + COL_BLK - 1) // COL_BLK

NC = 2
NS = 16
NW = NC * NS
BPW = BATCH // NW
IDX_ROWS = BPW // 128


def _lse_transpose_body(x_ref, lt_ref, lse_ref, m_acc, s_acc):
    i = pl.program_id(0)
    xt = x_ref[...].T
    row = i * COL_BLK + lax.broadcasted_iota(jnp.int32, (COL_BLK, NUM_STATES), 0)
    xt_m = jnp.where(row < NUM_OBS, xt, -jnp.inf)
    blk_max = jnp.max(xt_m, axis=0, keepdims=True)

    @pl.when(i == 0)
    def _():
        m_acc[...] = jnp.full((1, NUM_STATES), -jnp.inf, jnp.float32)
        s_acc[...] = jnp.zeros((1, NUM_STATES), jnp.float32)

    m_old = m_acc[...]
    m_new = jnp.maximum(m_old, blk_max)
    s_acc[...] = s_acc[...] * jnp.exp(m_old - m_new) + jnp.sum(
        jnp.exp(xt_m - m_new), axis=0, keepdims=True)
    m_acc[...] = m_new
    lt_ref[...] = xt

    @pl.when(i == NBLK - 1)
    def _():
        lse_ref[...] = m_acc[...] + jnp.log(s_acc[...])


_tc_pass = pl.pallas_call(
    _lse_transpose_body,
    grid=(NBLK,),
    in_specs=[pl.BlockSpec((NUM_STATES, COL_BLK), lambda i: (0, i))],
    out_specs=[
        pl.BlockSpec((COL_BLK, NUM_STATES), lambda i: (i, 0)),
        pl.BlockSpec((1, NUM_STATES), lambda i: (0, 0)),
    ],
    out_shape=[
        jax.ShapeDtypeStruct((NUM_OBS, NUM_STATES), jnp.float32),
        jax.ShapeDtypeStruct((1, NUM_STATES), jnp.float32),
    ],
    scratch_shapes=[
        pltpu.VMEM((1, NUM_STATES), jnp.float32),
        pltpu.VMEM((1, NUM_STATES), jnp.float32),
    ],
    compiler_params=pltpu.CompilerParams(
        dimension_semantics=("arbitrary",)),
)


def _sc_gather_body(table_hbm, obs_hbm, lse_hbm, out_hbm, idx_v, rows_v,
                    lse_v, sem):
    wid = lax.axis_index("s") * NC + lax.axis_index("c")
    base = wid * BPW
    pltpu.sync_copy(obs_hbm.at[wid], idx_v)
    pltpu.sync_copy(lse_hbm, lse_v)
    copies = [
        pltpu.async_copy(table_hbm.at[idx_v.at[j]],
                         rows_v.at[pl.ds(j * 128, 128)], sem)
        for j in range(IDX_ROWS)
    ]
    for c in copies:
        c.wait()

    lvecs = [lse_v[pl.ds(j * 16, 16)] for j in range(NUM_STATES // 16)]

    def body(r, carry):
        for j in range(NUM_STATES // 16):
            sl = pl.ds(j * 16, 16)
            rows_v[r, sl] = rows_v[r, sl] - lvecs[j]
        return carry

    lax.fori_loop(0, BPW, body, 0)
    pltpu.sync_copy(rows_v, out_hbm.at[pl.ds(base, BPW)])


@functools.cache
def _make_sc_gather():
    return pl.kernel(
        _sc_gather_body,
        out_type=jax.ShapeDtypeStruct((BATCH, NUM_STATES), jnp.float32),
        mesh=plsc.VectorSubcoreMesh(core_axis_name="c", subcore_axis_name="s",
                                    num_cores=NC, num_subcores=NS),
        scratch_types=[
            pltpu.VMEM((IDX_ROWS, 128), jnp.int32),
            pltpu.VMEM((BPW, NUM_STATES), jnp.float32),
            pltpu.VMEM((NUM_STATES,), jnp.float32),
            pltpu.SemaphoreType.DMA,
        ],
    )


def kernel(emission_logits_matrix, observation):
    obs = observation.astype(jnp.int32).reshape(NW, IDX_ROWS, 128)
    lt, lse = _tc_pass(emission_logits_matrix)
    return _make_sc_gather()(lt, obs, lse.reshape(NUM_STATES))


# speedup vs baseline: 1.7200x; 1.7200x over previous
import functools

import jax
import jax.numpy as jnp
from jax import lax
from jax.experimental import pallas as pl
from jax.experimental.pallas import tpu as pltpu
from jax.experimental.pallas import tpu_sc as plsc

NUM_STATES = 128
NUM_OBS = 100000
BATCH = 16384

COL_BLK = 4096
NBLK = (NUM_OBS + COL_BLK - 1) // COL_BLK

NC = 2
NS = 16
NW = NC * NS
BPW = BATCH // NW
IDX_ROWS = BPW // 128


def _lse_transpose_body(x_ref, lt_ref, lse_ref, m_acc, s_acc):
    i = pl.program_id(0)
    xt = x_ref[...].T
    lt_ref[...] = xt

    @pl.when(i == 0)
    def _():
        m_acc[...] = jnp.full((1, NUM_STATES), -jnp.inf, jnp.float32)
        s_acc[...] = jnp.zeros((1, NUM_STATES), jnp.float32)

    def update(xt_vals):
        m_old = m_acc[...]
        m_new = jnp.maximum(m_old, jnp.max(xt_vals, axis=0, keepdims=True))
        s_acc[...] = s_acc[...] * jnp.exp(m_old - m_new) + jnp.sum(
            jnp.exp(xt_vals - m_new), axis=0, keepdims=True)
        m_acc[...] = m_new

    @pl.when(i < NBLK - 1)
    def _():
        update(xt)

    @pl.when(i == NBLK - 1)
    def _():
        row = i * COL_BLK + lax.broadcasted_iota(
            jnp.int32, (COL_BLK, NUM_STATES), 0)
        update(jnp.where(row < NUM_OBS, xt, -jnp.inf))
        lse_ref[...] = m_acc[...] + jnp.log(s_acc[...])


_tc_pass = pl.pallas_call(
    _lse_transpose_body,
    grid=(NBLK,),
    in_specs=[pl.BlockSpec((NUM_STATES, COL_BLK), lambda i: (0, i))],
    out_specs=[
        pl.BlockSpec((COL_BLK, NUM_STATES), lambda i: (i, 0)),
        pl.BlockSpec((1, NUM_STATES), lambda i: (0, 0)),
    ],
    out_shape=[
        jax.ShapeDtypeStruct((NUM_OBS, NUM_STATES), jnp.float32),
        jax.ShapeDtypeStruct((1, NUM_STATES), jnp.float32),
    ],
    scratch_shapes=[
        pltpu.VMEM((1, NUM_STATES), jnp.float32),
        pltpu.VMEM((1, NUM_STATES), jnp.float32),
    ],
    compiler_params=pltpu.CompilerParams(
        dimension_semantics=("arbitrary",)),
)


def _sc_gather_body(table_hbm, obs_hbm, lse_hbm, out_hbm, idx_v, rows_v,
                    lse_v, sem):
    wid = lax.axis_index("s") * NC + lax.axis_index("c")
    base = wid * BPW
    pltpu.sync_copy(obs_hbm.at[wid], idx_v)
    pltpu.sync_copy(lse_hbm, lse_v)
    copies = [
        pltpu.async_copy(table_hbm.at[idx_v.at[j]],
                         rows_v.at[pl.ds(j * 128, 128)], sem)
        for j in range(IDX_ROWS)
    ]
    for c in copies:
        c.wait()

    lvecs = [lse_v[pl.ds(j * 16, 16)] for j in range(NUM_STATES // 16)]

    def body(r, carry):
        for j in range(NUM_STATES // 16):
            sl = pl.ds(j * 16, 16)
            rows_v[r, sl] = rows_v[r, sl] - lvecs[j]
        return carry

    lax.fori_loop(0, BPW, body, 0)
    pltpu.sync_copy(rows_v, out_hbm.at[pl.ds(base, BPW)])


@functools.cache
def _make_sc_gather():
    return pl.kernel(
        _sc_gather_body,
        out_type=jax.ShapeDtypeStruct((BATCH, NUM_STATES), jnp.float32),
        mesh=plsc.VectorSubcoreMesh(core_axis_name="c", subcore_axis_name="s",
                                    num_cores=NC, num_subcores=NS),
        scratch_types=[
            pltpu.VMEM((IDX_ROWS, 128), jnp.int32),
            pltpu.VMEM((BPW, NUM_STATES), jnp.float32),
            pltpu.VMEM((NUM_STATES,), jnp.float32),
            pltpu.SemaphoreType.DMA,
        ],
    )


def kernel(emission_logits_matrix, observation):
    obs = observation.astype(jnp.int32).reshape(NW, IDX_ROWS, 128)
    lt, lse = _tc_pass(emission_logits_matrix)
    return _make_sc_gather()(lt, obs, lse.reshape(NUM_STATES))


# speedup vs baseline: 1.8055x; 1.0497x over previous
import functools

import jax
import jax.numpy as jnp
from jax import lax
from jax.experimental import pallas as pl
from jax.experimental.pallas import tpu as pltpu
from jax.experimental.pallas import tpu_sc as plsc

NUM_STATES = 128
NUM_OBS = 100000
BATCH = 16384

COL_BLK = 8192
NBLK = (NUM_OBS + COL_BLK - 1) // COL_BLK

NC = 2
NS = 16
NW = NC * NS
BPW = BATCH // NW
IDX_ROWS = BPW // 128


def _lse_transpose_body(x_ref, lt_ref, lse_ref, m_acc, s_acc):
    i = pl.program_id(0)
    xt = x_ref[...].T
    lt_ref[...] = xt

    @pl.when(i == 0)
    def _():
        m_acc[...] = jnp.full((1, NUM_STATES), -jnp.inf, jnp.float32)
        s_acc[...] = jnp.zeros((1, NUM_STATES), jnp.float32)

    def update(xt_vals):
        m_old = m_acc[...]
        m_new = jnp.maximum(m_old, jnp.max(xt_vals, axis=0, keepdims=True))
        s_acc[...] = s_acc[...] * jnp.exp(m_old - m_new) + jnp.sum(
            jnp.exp(xt_vals - m_new), axis=0, keepdims=True)
        m_acc[...] = m_new

    @pl.when(i < NBLK - 1)
    def _():
        update(xt)

    @pl.when(i == NBLK - 1)
    def _():
        row = i * COL_BLK + lax.broadcasted_iota(
            jnp.int32, (COL_BLK, NUM_STATES), 0)
        update(jnp.where(row < NUM_OBS, xt, -jnp.inf))
        lse_ref[...] = m_acc[...] + jnp.log(s_acc[...])


_tc_pass = pl.pallas_call(
    _lse_transpose_body,
    grid=(NBLK,),
    in_specs=[pl.BlockSpec((NUM_STATES, COL_BLK), lambda i: (0, i))],
    out_specs=[
        pl.BlockSpec((COL_BLK, NUM_STATES), lambda i: (i, 0)),
        pl.BlockSpec((1, NUM_STATES), lambda i: (0, 0)),
    ],
    out_shape=[
        jax.ShapeDtypeStruct((NUM_OBS, NUM_STATES), jnp.float32),
        jax.ShapeDtypeStruct((1, NUM_STATES), jnp.float32),
    ],
    scratch_shapes=[
        pltpu.VMEM((1, NUM_STATES), jnp.float32),
        pltpu.VMEM((1, NUM_STATES), jnp.float32),
    ],
    compiler_params=pltpu.CompilerParams(
        dimension_semantics=("arbitrary",)),
)


def _sc_gather_body(table_hbm, obs_hbm, lse_hbm, out_hbm, idx_v, rows_v,
                    lse_v, sem):
    wid = lax.axis_index("s") * NC + lax.axis_index("c")
    base = wid * BPW
    pltpu.sync_copy(obs_hbm.at[wid], idx_v)
    pltpu.sync_copy(lse_hbm, lse_v)
    copies = [
        pltpu.async_copy(table_hbm.at[idx_v.at[j]],
                         rows_v.at[pl.ds(j * 128, 128)], sem)
        for j in range(IDX_ROWS)
    ]
    for c in copies:
        c.wait()

    lvecs = [lse_v[pl.ds(j * 16, 16)] for j in range(NUM_STATES // 16)]

    def body(r, carry):
        for j in range(NUM_STATES // 16):
            sl = pl.ds(j * 16, 16)
            rows_v[r, sl] = rows_v[r, sl] - lvecs[j]
        return carry

    lax.fori_loop(0, BPW, body, 0)
    pltpu.sync_copy(rows_v, out_hbm.at[pl.ds(base, BPW)])


@functools.cache
def _make_sc_gather():
    return pl.kernel(
        _sc_gather_body,
        out_type=jax.ShapeDtypeStruct((BATCH, NUM_STATES), jnp.float32),
        mesh=plsc.VectorSubcoreMesh(core_axis_name="c", subcore_axis_name="s",
                                    num_cores=NC, num_subcores=NS),
        scratch_types=[
            pltpu.VMEM((IDX_ROWS, 128), jnp.int32),
            pltpu.VMEM((BPW, NUM_STATES), jnp.float32),
            pltpu.VMEM((NUM_STATES,), jnp.float32),
            pltpu.SemaphoreType.DMA,
        ],
    )


def kernel(emission_logits_matrix, observation):
    obs = observation.astype(jnp.int32).reshape(NW, IDX_ROWS, 128)
    lt, lse = _tc_pass(emission_logits_matrix)
    return _make_sc_gather()(lt, obs, lse.reshape(NUM_STATES))
